# baseline (speedup 1.0000x reference)
import functools

import jax
import jax.numpy as jnp
from jax import lax
from jax.experimental import pallas as pl
from jax.experimental.pallas import tpu as pltpu
from jax.experimental.pallas import tpu_sc as plsc

N_REAL = 10000
H_PAD = 10240
D = 128
N_G = 128
E_REAL = 320000
NC, NS = 2, 16
NW = NC * NS
CK = 128
NCHUNK = 80
HALF = NCHUNK // 2
E_PAD = NW * NCHUNK * CK
PAD_NODE = N_REAL
ROWS_PER_TILE = H_PAD // NS
RB = 128


def _sc_agg_body(h_hbm, src_hbm, dst_hbm, out_hbm,
                 src_v, dst_v, rows0_v, rows1_v,
                 acc_sh, semg0, semg1, sems0, sems1):
    buf_v = rows0_v
    c = lax.axis_index("c")
    s = lax.axis_index("s")
    wid = c * NS + s

    zv = jnp.zeros((16,), jnp.float32)

    def zero_row(i, carry):
        for j in range(D // 16):
            buf_v[i, pl.ds(j * 16, 16)] = zv
        return carry

    lax.fori_loop(0, RB, zero_row, 0)
    for t in range(ROWS_PER_TILE // RB):
        pltpu.sync_copy(buf_v, acc_sh.at[pl.ds(s * ROWS_PER_TILE + t * RB, RB)])

    plsc.subcore_barrier()

    def g_start(j, buf, sem):
        pltpu.async_copy(h_hbm.at[src_v.at[j]], buf, sem)

    def g_wait(j, buf, sem):
        pltpu.make_async_copy(h_hbm.at[src_v.at[j]], buf, sem).wait()

    def s_start(j, buf, sem):
        pltpu.async_copy(buf, acc_sh.at[dst_v.at[j]], sem, add=True)

    def s_wait(j, buf, sem):
        pltpu.make_async_copy(buf, acc_sh.at[dst_v.at[j]], sem).wait()

    for phase in range(NCHUNK // HALF):
        pltpu.sync_copy(src_hbm.at[wid, pl.ds(phase * HALF, HALF)], src_v)
        pltpu.sync_copy(dst_hbm.at[wid, pl.ds(phase * HALF, HALF)], dst_v)

        g_start(0, rows0_v, semg0)
        g_start(1, rows1_v, semg1)

        def body(jj, carry):
            j = jj * 2
            g_wait(j, rows0_v, semg0)
            s_start(j, rows0_v, sems0)
            g_wait(j + 1, rows1_v, semg1)
            s_start(j + 1, rows1_v, sems1)
            s_wait(j, rows0_v, sems0)

            @pl.when(j + 2 < HALF)
            def _():
                g_start(j + 2, rows0_v, semg0)

            s_wait(j + 1, rows1_v, sems1)

            @pl.when(j + 3 < HALF)
            def _():
                g_start(j + 3, rows1_v, semg1)

            return carry

        lax.fori_loop(0, HALF // 2, body, 0)

    plsc.subcore_barrier()

    for t in range(ROWS_PER_TILE // RB):
        r0 = s * ROWS_PER_TILE + t * RB
        pltpu.sync_copy(acc_sh.at[pl.ds(r0, RB)], buf_v)
        pltpu.sync_copy(buf_v, out_hbm.at[c, pl.ds(r0, RB)])


_sc_agg = functools.partial(
    pl.kernel,
    out_type=jax.ShapeDtypeStruct((NC, H_PAD, D), jnp.float32),
    mesh=plsc.VectorSubcoreMesh(core_axis_name="c", subcore_axis_name="s"),
    scratch_types=[
        pltpu.VMEM((HALF, CK), jnp.int32),
        pltpu.VMEM((HALF, CK), jnp.int32),
        pltpu.VMEM((CK, D), jnp.float32),
        pltpu.VMEM((CK, D), jnp.float32),
        pltpu.VMEM_SHARED((H_PAD, D), jnp.float32),
        pltpu.SemaphoreType.DMA,
        pltpu.SemaphoreType.DMA,
        pltpu.SemaphoreType.DMA,
        pltpu.SemaphoreType.DMA,
    ],
)(_sc_agg_body)


BR = 1280


def _tc_layer_body(relu, a0_ref, a1_ref, h_ref, wr_ref, b_ref, wro_ref, o_ref):
    agg = a0_ref[...] + a1_ref[...]
    out = (jnp.dot(agg, wr_ref[...], preferred_element_type=jnp.float32)
           + b_ref[...]
           + jnp.dot(h_ref[...], wro_ref[...], preferred_element_type=jnp.float32))
    if relu:
        out = jnp.maximum(out, 0.0)
    o_ref[...] = out


def _tc_layer(a0, a1, h, w_rel, b, w_root, relu):
    return pl.pallas_call(
        functools.partial(_tc_layer_body, relu),
        grid=(H_PAD // BR,),
        in_specs=[
            pl.BlockSpec((BR, D), lambda i: (i, 0)),
            pl.BlockSpec((BR, D), lambda i: (i, 0)),
            pl.BlockSpec((BR, D), lambda i: (i, 0)),
            pl.BlockSpec((D, D), lambda i: (0, 0)),
            pl.BlockSpec((1, D), lambda i: (0, 0)),
            pl.BlockSpec((D, D), lambda i: (0, 0)),
        ],
        out_specs=pl.BlockSpec((BR, D), lambda i: (i, 0)),
        out_shape=jax.ShapeDtypeStruct((H_PAD, D), jnp.float32),
    )(a0, a1, h, w_rel, b.reshape(1, D), w_root)


def _tc_final_body(a0_ref, a1_ref, h_ref, wr_ref, b_ref, wro_ref,
                   batch_ref, wl_ref, bl_ref, o_ref, sums_v, counts_v):
    i = pl.program_id(0)

    @pl.when(i == 0)
    def _init():
        sums_v[...] = jnp.zeros((N_G, D), jnp.float32)
        counts_v[...] = jnp.zeros((N_G, D), jnp.float32)

    agg = a0_ref[...] + a1_ref[...]
    h3 = (jnp.dot(agg, wr_ref[...], preferred_element_type=jnp.float32)
          + b_ref[...]
          + jnp.dot(h_ref[...], wro_ref[...], preferred_element_type=jnp.float32))
    onehot = (batch_ref[...] == lax.broadcasted_iota(jnp.int32, (1, N_G), 1)
              ).astype(jnp.float32)
    sums_v[...] += lax.dot_general(onehot, h3, (((0,), (0,)), ((), ())),
                                   preferred_element_type=jnp.float32)
    counts_v[...] += lax.dot_general(onehot, jnp.ones((BR, D), jnp.float32),
                                     (((0,), (0,)), ((), ())),
                                     preferred_element_type=jnp.float32)

    @pl.when(i == pl.num_programs(0) - 1)
    def _fin():
        pooled = sums_v[...] / jnp.maximum(counts_v[...], 1.0)
        o_ref[...] = (jnp.dot(pooled, wl_ref[...],
                              preferred_element_type=jnp.float32) + bl_ref[...])


def _tc_final(a0, a1, h, w_rel, b, w_root, batch2d, w_lin_pad, b_lin_pad):
    return pl.pallas_call(
        _tc_final_body,
        grid=(H_PAD // BR,),
        in_specs=[
            pl.BlockSpec((BR, D), lambda i: (i, 0)),
            pl.BlockSpec((BR, D), lambda i: (i, 0)),
            pl.BlockSpec((BR, D), lambda i: (i, 0)),
            pl.BlockSpec((D, D), lambda i: (0, 0)),
            pl.BlockSpec((1, D), lambda i: (0, 0)),
            pl.BlockSpec((D, D), lambda i: (0, 0)),
            pl.BlockSpec((BR, 1), lambda i: (i, 0)),
            pl.BlockSpec((D, D), lambda i: (0, 0)),
            pl.BlockSpec((1, D), lambda i: (0, 0)),
        ],
        out_specs=pl.BlockSpec((N_G, D), lambda i: (0, 0)),
        out_shape=jax.ShapeDtypeStruct((N_G, D), jnp.float32),
        scratch_shapes=[
            pltpu.VMEM((N_G, D), jnp.float32),
            pltpu.VMEM((N_G, D), jnp.float32),
        ],
    )(a0, a1, h, w_rel, b.reshape(1, D), w_root, batch2d, w_lin_pad, b_lin_pad)


def kernel(x, edge_index, batch, W1_rel, b1, W1_root, W2_rel, b2, W2_root,
           W3_rel, b3, W3_root, W_lin, b_lin):
    src = edge_index[0].astype(jnp.int32)
    dst = edge_index[1].astype(jnp.int32)
    pad = E_PAD - E_REAL
    pad_idx = jnp.arange(pad, dtype=jnp.int32) % (H_PAD - N_REAL) + N_REAL
    src3 = jnp.concatenate([src, pad_idx]).reshape(NW, NCHUNK, CK)
    dst3 = jnp.concatenate([dst, pad_idx]).reshape(NW, NCHUNK, CK)
    h0 = jnp.pad(x, ((0, H_PAD - N_REAL), (0, 0)))
    batch2d = jnp.pad(batch.astype(jnp.int32), (0, H_PAD - N_REAL),
                      constant_values=N_G).reshape(H_PAD, 1)
    w_lin_pad = jnp.pad(W_lin, ((0, 0), (0, D - W_lin.shape[1])))
    b_lin_pad = jnp.pad(b_lin, (0, D - b_lin.shape[0])).reshape(1, D)

    aggp = _sc_agg(h0, src3, dst3)
    h1 = _tc_layer(aggp[0], aggp[1], h0, W1_rel, b1, W1_root, relu=True)
    aggp = _sc_agg(h1, src3, dst3)
    h2 = _tc_layer(aggp[0], aggp[1], h1, W2_rel, b2, W2_root, relu=True)
    aggp = _sc_agg(h2, src3, dst3)
    out = _tc_final(aggp[0], aggp[1], h2, W3_rel, b3, W3_root,
                    batch2d, w_lin_pad, b_lin_pad)
    return out[:, :2]


# speedup vs baseline: 1.1523x; 1.1523x over previous
import functools

import jax
import jax.numpy as jnp
from jax import lax
from jax.experimental import pallas as pl
from jax.experimental.pallas import tpu as pltpu
from jax.experimental.pallas import tpu_sc as plsc

N_REAL = 10000
H_PAD = 10240
D = 128
N_G = 128
E_REAL = 320000
NC, NS = 2, 16
NW = NC * NS
CK = 64
NCHUNK = 160
PH = 4
PCH = NCHUNK // PH
E_PAD = NW * NCHUNK * CK
PAD_NODE = N_REAL
ROWS_PER_TILE = H_PAD // NS
RB = 128


def _sc_agg_body(h_hbm, src_hbm, dst_hbm, out_hbm,
                 src_v, dst_v, rows0_v, rows1_v, rows2_v, rows3_v,
                 acc_sh, semg0, semg1, semg2, semg3, sems0, sems1, sems2, sems3):
    buf_v = rows0_v
    c = lax.axis_index("c")
    s = lax.axis_index("s")
    wid = c * NS + s

    zv = jnp.zeros((16,), jnp.float32)

    def zero_row(i, carry):
        for j in range(D // 16):
            buf_v[i, pl.ds(j * 16, 16)] = zv
        return carry

    lax.fori_loop(0, CK, zero_row, 0)
    for t in range(ROWS_PER_TILE // CK):
        pltpu.sync_copy(buf_v, acc_sh.at[pl.ds(s * ROWS_PER_TILE + t * CK, CK)])

    plsc.subcore_barrier()

    def g_start(j, buf, sem):
        pltpu.async_copy(h_hbm.at[src_v.at[j]], buf, sem)

    def g_wait(j, buf, sem):
        pltpu.make_async_copy(h_hbm.at[src_v.at[j]], buf, sem).wait()

    def s_start(j, buf, sem):
        pltpu.async_copy(buf, acc_sh.at[dst_v.at[j]], sem, add=True)

    def s_wait(j, buf, sem):
        pltpu.make_async_copy(buf, acc_sh.at[dst_v.at[j]], sem).wait()

    for phase in range(PH):
        pltpu.sync_copy(src_hbm.at[wid, pl.ds(phase * PCH, PCH)], src_v)
        pltpu.sync_copy(dst_hbm.at[wid, pl.ds(phase * PCH, PCH)], dst_v)

        bufs = ((rows0_v, semg0, sems0), (rows1_v, semg1, sems1),
                (rows2_v, semg2, sems2), (rows3_v, semg3, sems3))
        for b, (buf, sg, _) in enumerate(bufs):
            g_start(b, buf, sg)

        nd = len(bufs)

        def body(jj, carry):
            j = jj * nd
            for b, (buf, sg, ss) in enumerate(bufs):
                g_wait(j + b, buf, sg)
                s_start(j + b, buf, ss)
            for b, (buf, sg, ss) in enumerate(bufs):
                s_wait(j + b, buf, ss)

                @pl.when(j + b + nd < PCH)
                def _(buf=buf, sg=sg, jn=j + b + nd):
                    g_start(jn, buf, sg)

            return carry

        lax.fori_loop(0, PCH // nd, body, 0)

    plsc.subcore_barrier()

    for t in range(ROWS_PER_TILE // CK):
        r0 = s * ROWS_PER_TILE + t * CK
        pltpu.sync_copy(acc_sh.at[pl.ds(r0, CK)], buf_v)
        pltpu.sync_copy(buf_v, out_hbm.at[c, pl.ds(r0, CK)])


_sc_agg = functools.partial(
    pl.kernel,
    out_type=jax.ShapeDtypeStruct((NC, H_PAD, D), jnp.float32),
    mesh=plsc.VectorSubcoreMesh(core_axis_name="c", subcore_axis_name="s"),
    scratch_types=[
        pltpu.VMEM((PCH, CK), jnp.int32),
        pltpu.VMEM((PCH, CK), jnp.int32),
        pltpu.VMEM((CK, D), jnp.float32),
        pltpu.VMEM((CK, D), jnp.float32),
        pltpu.VMEM((CK, D), jnp.float32),
        pltpu.VMEM((CK, D), jnp.float32),
        pltpu.VMEM_SHARED((H_PAD, D), jnp.float32),
        pltpu.SemaphoreType.DMA,
        pltpu.SemaphoreType.DMA,
        pltpu.SemaphoreType.DMA,
        pltpu.SemaphoreType.DMA,
        pltpu.SemaphoreType.DMA,
        pltpu.SemaphoreType.DMA,
        pltpu.SemaphoreType.DMA,
        pltpu.SemaphoreType.DMA,
    ],
)(_sc_agg_body)


BR = 1280


def _tc_layer_body(relu, a0_ref, a1_ref, h_ref, wr_ref, b_ref, wro_ref, o_ref):
    agg = a0_ref[...] + a1_ref[...]
    out = (jnp.dot(agg, wr_ref[...], preferred_element_type=jnp.float32)
           + b_ref[...]
           + jnp.dot(h_ref[...], wro_ref[...], preferred_element_type=jnp.float32))
    if relu:
        out = jnp.maximum(out, 0.0)
    o_ref[...] = out


def _tc_layer(a0, a1, h, w_rel, b, w_root, relu):
    return pl.pallas_call(
        functools.partial(_tc_layer_body, relu),
        grid=(H_PAD // BR,),
        in_specs=[
            pl.BlockSpec((BR, D), lambda i: (i, 0)),
            pl.BlockSpec((BR, D), lambda i: (i, 0)),
            pl.BlockSpec((BR, D), lambda i: (i, 0)),
            pl.BlockSpec((D, D), lambda i: (0, 0)),
            pl.BlockSpec((1, D), lambda i: (0, 0)),
            pl.BlockSpec((D, D), lambda i: (0, 0)),
        ],
        out_specs=pl.BlockSpec((BR, D), lambda i: (i, 0)),
        out_shape=jax.ShapeDtypeStruct((H_PAD, D), jnp.float32),
    )(a0, a1, h, w_rel, b.reshape(1, D), w_root)


def _tc_final_body(a0_ref, a1_ref, h_ref, wr_ref, b_ref, wro_ref,
                   batch_ref, wl_ref, bl_ref, o_ref, sums_v, counts_v):
    i = pl.program_id(0)

    @pl.when(i == 0)
    def _init():
        sums_v[...] = jnp.zeros((N_G, D), jnp.float32)
        counts_v[...] = jnp.zeros((N_G, D), jnp.float32)

    agg = a0_ref[...] + a1_ref[...]
    h3 = (jnp.dot(agg, wr_ref[...], preferred_element_type=jnp.float32)
          + b_ref[...]
          + jnp.dot(h_ref[...], wro_ref[...], preferred_element_type=jnp.float32))
    onehot = (batch_ref[...] == lax.broadcasted_iota(jnp.int32, (1, N_G), 1)
              ).astype(jnp.float32)
    sums_v[...] += lax.dot_general(onehot, h3, (((0,), (0,)), ((), ())),
                                   preferred_element_type=jnp.float32)
    counts_v[...] += lax.dot_general(onehot, jnp.ones((BR, D), jnp.float32),
                                     (((0,), (0,)), ((), ())),
                                     preferred_element_type=jnp.float32)

    @pl.when(i == pl.num_programs(0) - 1)
    def _fin():
        pooled = sums_v[...] / jnp.maximum(counts_v[...], 1.0)
        o_ref[...] = (jnp.dot(pooled, wl_ref[...],
                              preferred_element_type=jnp.float32) + bl_ref[...])


def _tc_final(a0, a1, h, w_rel, b, w_root, batch2d, w_lin_pad, b_lin_pad):
    return pl.pallas_call(
        _tc_final_body,
        grid=(H_PAD // BR,),
        in_specs=[
            pl.BlockSpec((BR, D), lambda i: (i, 0)),
            pl.BlockSpec((BR, D), lambda i: (i, 0)),
            pl.BlockSpec((BR, D), lambda i: (i, 0)),
            pl.BlockSpec((D, D), lambda i: (0, 0)),
            pl.BlockSpec((1, D), lambda i: (0, 0)),
            pl.BlockSpec((D, D), lambda i: (0, 0)),
            pl.BlockSpec((BR, 1), lambda i: (i, 0)),
            pl.BlockSpec((D, D), lambda i: (0, 0)),
            pl.BlockSpec((1, D), lambda i: (0, 0)),
        ],
        out_specs=pl.BlockSpec((N_G, D), lambda i: (0, 0)),
        out_shape=jax.ShapeDtypeStruct((N_G, D), jnp.float32),
        scratch_shapes=[
            pltpu.VMEM((N_G, D), jnp.float32),
            pltpu.VMEM((N_G, D), jnp.float32),
        ],
    )(a0, a1, h, w_rel, b.reshape(1, D), w_root, batch2d, w_lin_pad, b_lin_pad)


def kernel(x, edge_index, batch, W1_rel, b1, W1_root, W2_rel, b2, W2_root,
           W3_rel, b3, W3_root, W_lin, b_lin):
    src = edge_index[0].astype(jnp.int32)
    dst = edge_index[1].astype(jnp.int32)
    pad = E_PAD - E_REAL
    pad_idx = jnp.arange(pad, dtype=jnp.int32) % (H_PAD - N_REAL) + N_REAL
    src3 = jnp.concatenate([src, pad_idx]).reshape(NW, NCHUNK, CK)
    dst3 = jnp.concatenate([dst, pad_idx]).reshape(NW, NCHUNK, CK)
    h0 = jnp.pad(x, ((0, H_PAD - N_REAL), (0, 0)))
    batch2d = jnp.pad(batch.astype(jnp.int32), (0, H_PAD - N_REAL),
                      constant_values=N_G).reshape(H_PAD, 1)
    w_lin_pad = jnp.pad(W_lin, ((0, 0), (0, D - W_lin.shape[1])))
    b_lin_pad = jnp.pad(b_lin, (0, D - b_lin.shape[0])).reshape(1, D)

    aggp = _sc_agg(h0, src3, dst3)
    h1 = _tc_layer(aggp[0], aggp[1], h0, W1_rel, b1, W1_root, relu=True)
    aggp = _sc_agg(h1, src3, dst3)
    h2 = _tc_layer(aggp[0], aggp[1], h1, W2_rel, b2, W2_root, relu=True)
    aggp = _sc_agg(h2, src3, dst3)
    out = _tc_final(aggp[0], aggp[1], h2, W3_rel, b3, W3_root,
                    batch2d, w_lin_pad, b_lin_pad)
    return out[:, :2]
